# baseline (speedup 1.0000x reference)
import functools

import jax
import jax.numpy as jnp
from jax import lax
from jax.experimental import pallas as pl
from jax.experimental.pallas import tpu as pltpu
from jax.experimental.pallas import tpu_sc as plsc

N = 10000
B = 256
H = 10
C = 77
E_RAW = 160000
E_TOT = E_RAW + N
NP = 10240
EP = 172032
CHUNK = 128
TPA = EP // 32
TPBC = EP // 16
NCH_A = TPA // CHUNK
NCH_BC = TPBC // CHUNK
RPT = NP // 16

_MESH = plsc.VectorSubcoreMesh(core_axis_name="c", subcore_axis_name="s")


def _rup(x, m):
    return (x + m - 1) // m * m


def _mm_body(x_ref, w_ref, b_ref, o_ref, *, act):
    acc = jnp.dot(x_ref[...], w_ref[...], preferred_element_type=jnp.float32)
    acc = acc + b_ref[...]
    if act == "relu":
        acc = jnp.maximum(acc, 0.0)
    elif act == "elu":
        acc = jnp.where(acc > 0, acc, jnp.exp(acc) - 1.0)
    o_ref[...] = acc


def _mm(x, w, b, act="none", bm=1024, bn=1024):
    M, K = x.shape
    _, Nn = w.shape
    bm = min(bm, _rup(M, 8))
    bn = min(bn, _rup(Nn, 128))
    Mp, Kp, Np2 = _rup(M, bm), _rup(K, 128), _rup(Nn, bn)
    xp = jnp.pad(x, ((0, Mp - M), (0, Kp - K)))
    wp = jnp.pad(w, ((0, Kp - K), (0, Np2 - Nn)))
    bp = jnp.pad(b.reshape(1, -1), ((0, 0), (0, Np2 - Nn)))
    out = pl.pallas_call(
        functools.partial(_mm_body, act=act),
        grid=(Mp // bm, Np2 // bn),
        in_specs=[
            pl.BlockSpec((bm, Kp), lambda i, j: (i, 0)),
            pl.BlockSpec((Kp, bn), lambda i, j: (0, j)),
            pl.BlockSpec((1, bn), lambda i, j: (0, j)),
        ],
        out_specs=pl.BlockSpec((bm, bn), lambda i, j: (i, j)),
        out_shape=jax.ShapeDtypeStruct((Mp, Np2), jnp.float32),
    )(xp, wp, bp)
    return out[:M, :Nn]


def _edge_softmax_body(src_hbm, dst_hbm, es_hbm, ed_hbm,
                       ex_hbm, den0_hbm, den1_hbm,
                       sidx0, sidx1, didx0, didx1, esr0, esr1, edr0, edr1,
                       exb, zb, acc, si, sg):
    c = lax.axis_index("c")
    s = lax.axis_index("s")
    w = s * 2 + c
    lane = lax.iota(jnp.int32, 16)
    SIDX = (sidx0, sidx1)
    DIDX = (didx0, didx1)
    ESR = (esr0, esr1)
    EDR = (edr0, edr1)

    def idx_start(b, chv):
        base = w * TPA + chv * CHUNK
        pltpu.async_copy(src_hbm.at[pl.ds(base, CHUNK)], SIDX[b], si)
        pltpu.async_copy(dst_hbm.at[pl.ds(base, CHUNK)], DIDX[b], si)

    def idx_drain(b):
        pltpu.make_async_copy(src_hbm.at[pl.ds(0, CHUNK)], SIDX[b], si).wait()
        pltpu.make_async_copy(dst_hbm.at[pl.ds(0, CHUNK)], DIDX[b], si).wait()

    def gather_start(b):
        pltpu.async_copy(es_hbm.at[SIDX[b]], ESR[b], sg)
        pltpu.async_copy(ed_hbm.at[DIDX[b]], EDR[b], sg)

    def gather_drain(b):
        pltpu.make_async_copy(es_hbm.at[pl.ds(0, CHUNK)], ESR[b], sg).wait()
        pltpu.make_async_copy(ed_hbm.at[pl.ds(0, CHUNK)], EDR[b], sg).wait()

    def zrow(i, _):
        zb[i, :] = jnp.zeros((16,), jnp.float32)
        return 0
    lax.fori_loop(0, 16, zrow, 0)

    def zcp(j, _):
        pltpu.sync_copy(zb, acc.at[pl.ds(s * RPT + j * 16, 16)])
        return 0
    lax.fori_loop(0, RPT // 16, zcp, 0)
    plsc.subcore_barrier()

    def compute_scatter(b, chv):
        base = w * TPA + chv * CHUNK
        esr_b = ESR[b]
        edr_b = EDR[b]

        def edge(k, _):
            v = esr_b[k, :] + edr_b[k, :]
            v = jnp.maximum(v, 0.2 * v)
            v = jnp.exp(v)
            v = jnp.where(lane == 10, 1.0, v)
            exb[k, :] = v
            return 0
        lax.fori_loop(0, CHUNK, edge, 0, unroll=4)
        pltpu.sync_copy(exb, ex_hbm.at[pl.ds(base, CHUNK)])
        pltpu.sync_copy(exb, acc.at[DIDX[b]], add=True)

    idx_start(0, 0)
    idx_drain(0)
    gather_start(0)
    idx_start(1, 1)

    def grp(i, _):
        nxt = jnp.where(2 * i + 2 < NCH_A, 2 * i + 2, 0)
        nxt2 = jnp.where(2 * i + 3 < NCH_A, 2 * i + 3, 0)
        idx_drain(1)
        gather_drain(0)
        gather_start(1)
        compute_scatter(0, 2 * i)
        idx_start(0, nxt)
        idx_drain(0)
        gather_drain(1)
        gather_start(0)
        compute_scatter(1, 2 * i + 1)
        idx_start(1, nxt2)
        return 0
    lax.fori_loop(0, NCH_A // 2, grp, 0)
    idx_drain(1)
    gather_drain(0)
    plsc.subcore_barrier()

    @pl.when(c == 0)
    def _():
        pltpu.sync_copy(acc.at[pl.ds(s * RPT, RPT)],
                        den0_hbm.at[pl.ds(s * RPT, RPT)])

    @pl.when(c == 1)
    def _():
        pltpu.sync_copy(acc.at[pl.ds(s * RPT, RPT)],
                        den1_hbm.at[pl.ds(s * RPT, RPT)])


_edge_softmax = pl.kernel(
    _edge_softmax_body,
    out_type=(
        jax.ShapeDtypeStruct((EP, 16), jnp.float32),
        jax.ShapeDtypeStruct((NP, 16), jnp.float32),
        jax.ShapeDtypeStruct((NP, 16), jnp.float32),
    ),
    mesh=_MESH,
    compiler_params=pltpu.CompilerParams(use_tc_tiling_on_sc=False, needs_layout_passes=False),
    scratch_types=(
        pltpu.VMEM((CHUNK,), jnp.int32),
        pltpu.VMEM((CHUNK,), jnp.int32),
        pltpu.VMEM((CHUNK,), jnp.int32),
        pltpu.VMEM((CHUNK,), jnp.int32),
        pltpu.VMEM((CHUNK, 16), jnp.float32),
        pltpu.VMEM((CHUNK, 16), jnp.float32),
        pltpu.VMEM((CHUNK, 16), jnp.float32),
        pltpu.VMEM((CHUNK, 16), jnp.float32),
        pltpu.VMEM((CHUNK, 16), jnp.float32),
        pltpu.VMEM((16, 16), jnp.float32),
        pltpu.MemorySpace.VMEM_SHARED(shape=(NP, 16), dtype=jnp.float32),
        pltpu.SemaphoreType.DMA,
        pltpu.SemaphoreType.DMA,
    ),
)


def _gat_agg_body(src_hbm, dst_hbm, x_hbm, ex_hbm, agg_hbm,
                  sidx0, sidx1, didx0, didx1, xr0, xr1, exb0, exb1,
                  zb, acc, si, sg):
    c = lax.axis_index("c")
    s = lax.axis_index("s")
    SIDX = (sidx0, sidx1)
    DIDX = (didx0, didx1)
    XR = (xr0, xr1)
    EXB = (exb0, exb1)
    SI = (si, si)
    SG = (sg, sg)

    def idx_start(b, chv):
        base = s * TPBC + chv * CHUNK
        pltpu.async_copy(src_hbm.at[pl.ds(base, CHUNK)], SIDX[b], SI[b])
        pltpu.async_copy(dst_hbm.at[pl.ds(base, CHUNK)], DIDX[b], SI[b])
        pltpu.async_copy(ex_hbm.at[pl.ds(base, CHUNK)], EXB[b], SI[b])

    def idx_drain(b):
        pltpu.make_async_copy(src_hbm.at[pl.ds(0, CHUNK)], SIDX[b], SI[b]).wait()
        pltpu.make_async_copy(dst_hbm.at[pl.ds(0, CHUNK)], DIDX[b], SI[b]).wait()
        pltpu.make_async_copy(ex_hbm.at[pl.ds(0, CHUNK)], EXB[b], SI[b]).wait()

    def gather_start(b):
        pltpu.async_copy(x_hbm.at[SIDX[b]], XR[b], SG[b])

    def gather_drain(b):
        pltpu.make_async_copy(x_hbm.at[pl.ds(0, CHUNK)], XR[b], SG[b]).wait()

    def zrow(i, _):
        for g in range(5):
            zb[i, pl.ds(16 * g, 16)] = jnp.zeros((16,), jnp.float32)
        return 0
    lax.fori_loop(0, 16, zrow, 0)

    for p in range(5):
        def zcp(j, _):
            pltpu.sync_copy(zb, acc.at[pl.ds(s * RPT + j * 16, 16)])
            return 0
        lax.fori_loop(0, RPT // 16, zcp, 0)
        plsc.subcore_barrier()
        hl = c * 5 + p
        hlv = jnp.full((16,), hl, jnp.int32)

        def compute_scatter(b, hlv=hlv):
            xr_b = XR[b]
            exb_b = EXB[b]

            def edge(k, _):
                kv = jnp.full((16,), k, jnp.int32)
                wv = plsc.load_gather(exb_b, [kv, hlv])
                for g in range(5):
                    xr_b[k, pl.ds(16 * g, 16)] = xr_b[k, pl.ds(16 * g, 16)] * wv
                return 0
            lax.fori_loop(0, CHUNK, edge, 0, unroll=4)
            pltpu.sync_copy(xr_b, acc.at[DIDX[b]], add=True)

        idx_start(0, 0)
        idx_drain(0)
        gather_start(0)
        idx_start(1, 1)

        def grp(i, _):
            nxt = jnp.where(2 * i + 2 < NCH_BC, 2 * i + 2, 0)
            nxt2 = jnp.where(2 * i + 3 < NCH_BC, 2 * i + 3, 0)
            idx_drain(1)
            gather_drain(0)
            gather_start(1)
            compute_scatter(0)
            idx_start(0, nxt)
            idx_drain(0)
            gather_drain(1)
            gather_start(0)
            compute_scatter(1)
            idx_start(1, nxt2)
            return 0
        lax.fori_loop(0, NCH_BC // 2, grp, 0)
        idx_drain(1)
        gather_drain(0)
        plsc.subcore_barrier()

        @pl.when(c == 0)
        def _(p=p):
            pltpu.sync_copy(acc.at[pl.ds(s * RPT, RPT)],
                            agg_hbm.at[p, pl.ds(s * RPT, RPT)])

        @pl.when(c == 1)
        def _(p=p):
            pltpu.sync_copy(acc.at[pl.ds(s * RPT, RPT)],
                            agg_hbm.at[5 + p, pl.ds(s * RPT, RPT)])
        plsc.subcore_barrier()


_gat_agg = pl.kernel(
    _gat_agg_body,
    out_type=jax.ShapeDtypeStruct((H, NP, 80), jnp.float32),
    mesh=_MESH,
    compiler_params=pltpu.CompilerParams(use_tc_tiling_on_sc=False, needs_layout_passes=False),
    scratch_types=(
        pltpu.VMEM((CHUNK,), jnp.int32),
        pltpu.VMEM((CHUNK,), jnp.int32),
        pltpu.VMEM((CHUNK,), jnp.int32),
        pltpu.VMEM((CHUNK,), jnp.int32),
        pltpu.VMEM((CHUNK, 80), jnp.float32),
        pltpu.VMEM((CHUNK, 80), jnp.float32),
        pltpu.VMEM((CHUNK, 16), jnp.float32),
        pltpu.VMEM((CHUNK, 16), jnp.float32),
        pltpu.VMEM((16, 80), jnp.float32),
        pltpu.MemorySpace.VMEM_SHARED(shape=(NP, 80), dtype=jnp.float32),
        pltpu.SemaphoreType.DMA,
        pltpu.SemaphoreType.DMA,
    ),
)


def _gcn_agg_body(src_hbm, dst_hbm, h2_hbm, agg_hbm,
                  sidx0, sidx1, didx0, didx1, r0, r1, zb, acc,
                  si, sg):
    c = lax.axis_index("c")
    s = lax.axis_index("s")
    SIDX = (sidx0, sidx1)
    DIDX = (didx0, didx1)
    RR = (r0, r1)
    SI = (si, si)
    SG = (sg, sg)

    def idx_start(b, chv):
        base = s * TPBC + chv * CHUNK
        pltpu.async_copy(src_hbm.at[pl.ds(base, CHUNK)], SIDX[b], SI[b])
        pltpu.async_copy(dst_hbm.at[pl.ds(base, CHUNK)], DIDX[b], SI[b])

    def idx_drain_fix(b, blk):
        pltpu.make_async_copy(src_hbm.at[pl.ds(0, CHUNK)], SIDX[b], SI[b]).wait()
        pltpu.make_async_copy(dst_hbm.at[pl.ds(0, CHUNK)], DIDX[b], SI[b]).wait()
        if blk:
            for g in range(CHUNK // 16):
                SIDX[b][pl.ds(16 * g, 16)] = (
                    SIDX[b][pl.ds(16 * g, 16)] + blk * NP)

    def gather_start(b):
        pltpu.async_copy(h2_hbm.at[SIDX[b]], RR[b], SG[b])

    def gather_drain(b):
        pltpu.make_async_copy(h2_hbm.at[pl.ds(0, CHUNK)], RR[b], SG[b]).wait()

    def scat(b):
        pltpu.sync_copy(RR[b], acc.at[DIDX[b]], add=True)

    def zrow(i, _):
        for g in range(5):
            zb[i, pl.ds(16 * g, 16)] = jnp.zeros((16,), jnp.float32)
        return 0
    lax.fori_loop(0, 16, zrow, 0)

    for blk in range(H):
        owner = blk % 2

        @pl.when(c == owner)
        def _(blk=blk):
            def zcp(j, _):
                pltpu.sync_copy(zb, acc.at[pl.ds(s * RPT + j * 16, 16)])
                return 0
            lax.fori_loop(0, RPT // 16, zcp, 0)
            plsc.subcore_barrier()
            idx_start(0, 0)
            idx_drain_fix(0, blk)
            gather_start(0)
            idx_start(1, 1)

            def grp(i, _, blk=blk):
                nxt = jnp.where(2 * i + 2 < NCH_BC, 2 * i + 2, 0)
                nxt2 = jnp.where(2 * i + 3 < NCH_BC, 2 * i + 3, 0)
                idx_drain_fix(1, blk)
                gather_drain(0)
                gather_start(1)
                scat(0)
                idx_start(0, nxt)
                idx_drain_fix(0, blk)
                gather_drain(1)
                gather_start(0)
                scat(1)
                idx_start(1, nxt2)
                return 0
            lax.fori_loop(0, NCH_BC // 2, grp, 0)
            pltpu.make_async_copy(src_hbm.at[pl.ds(0, CHUNK)], sidx1, si).wait()
            pltpu.make_async_copy(dst_hbm.at[pl.ds(0, CHUNK)], didx1, si).wait()
            gather_drain(0)
            plsc.subcore_barrier()
            pltpu.sync_copy(acc.at[pl.ds(s * RPT, RPT)],
                            agg_hbm.at[blk, pl.ds(s * RPT, RPT)])
            plsc.subcore_barrier()


_gcn_agg = pl.kernel(
    _gcn_agg_body,
    out_type=jax.ShapeDtypeStruct((H, NP, 80), jnp.float32),
    mesh=_MESH,
    compiler_params=pltpu.CompilerParams(use_tc_tiling_on_sc=False, needs_layout_passes=False),
    scratch_types=(
        pltpu.VMEM((CHUNK,), jnp.int32),
        pltpu.VMEM((CHUNK,), jnp.int32),
        pltpu.VMEM((CHUNK,), jnp.int32),
        pltpu.VMEM((CHUNK,), jnp.int32),
        pltpu.VMEM((CHUNK, 80), jnp.float32),
        pltpu.VMEM((CHUNK, 80), jnp.float32),
        pltpu.VMEM((16, 80), jnp.float32),
        pltpu.MemorySpace.VMEM_SHARED(shape=(NP, 80), dtype=jnp.float32),
        pltpu.SemaphoreType.DMA,
        pltpu.SemaphoreType.DMA,
    ),
)


def _branch(x, edge_index, batch, Wg, a_s, a_d, bg, Wc, bc, W1, b1, W2, b2):
    loop = jnp.arange(N, dtype=jnp.int32)
    padi = jnp.full((EP - E_TOT,), N, jnp.int32)
    src = jnp.concatenate([edge_index[0].astype(jnp.int32), loop, padi])
    dst = jnp.concatenate([edge_index[1].astype(jnp.int32), loop, padi])

    W3 = Wg.reshape(C, H, C)
    As = jnp.einsum('chd,hd->ch', W3, a_s)
    Ad = jnp.einsum('chd,hd->ch', W3, a_d)
    AsAd = jnp.concatenate([
        jnp.pad(As, ((0, 0), (0, 6))), jnp.pad(Ad, ((0, 0), (0, 6)))], axis=1)
    esed = _mm(x, AsAd, jnp.zeros((32,), jnp.float32))
    es = jnp.pad(esed[:, :16], ((0, NP - N), (0, 0)))
    ed = jnp.pad(esed[:, 16:], ((0, NP - N), (0, 0)))

    ex, den0, den1 = _edge_softmax(src, dst, es, ed)
    den = den0[:N] + den1[:N]
    deg = den[:, 10]
    dinv = deg ** -0.5

    xpad = jnp.pad(x, ((0, NP - N), (0, 80 - C)))
    agg = _gat_agg(src, dst, xpad, ex)
    aggt = agg[:, :N, :].transpose(1, 0, 2) / (den[:, :10, None] + 1e-16)
    W_bd = jax.scipy.linalg.block_diag(
        *[jnp.pad(W3[:, h, :], ((0, 3), (0, 0))) for h in range(H)])
    gat = _mm(aggt.reshape(N, H * 80), W_bd, bg, act="elu")

    y = gat * dinv[:, None]
    h2 = _mm(y, Wc, jnp.zeros((H * C,), jnp.float32))
    h2r = jnp.pad(h2, ((0, NP - N), (0, 30))).reshape(NP, H, 80).transpose(1, 0, 2)
    aggc = _gcn_agg(src, dst, h2r.reshape(H * NP, 80))
    back = aggc.transpose(1, 0, 2).reshape(NP, H * 80)[:N, :H * C]
    z = jnp.maximum(back * dinv[:, None] + bc, 0.0)

    pooled = _pool(z, batch)
    z = _mm(pooled, W1, b1, act="relu")
    z = _mm(z, W2, b2, act="relu")
    return z


def _pool(x, batch):
    batch = batch.astype(jnp.int32)
    mx = jax.ops.segment_max(x, batch, num_segments=B)
    mx = jnp.where(jnp.isfinite(mx), mx, 0.0)
    x1 = jnp.concatenate([x, jnp.ones((x.shape[0], 1), x.dtype)], axis=1)
    pt = (jnp.arange(B, dtype=jnp.int32)[:, None] == batch[None, :]).astype(jnp.float32)
    kc = 2500
    se = sum(_mm(pt[:, k:k + kc], x1[k:k + kc],
                 jnp.zeros((x1.shape[1],), jnp.float32))
             for k in range(0, N, kc))
    cnt = se[:, -1]
    mean = se[:, :-1] / jnp.maximum(cnt, 1.0)[:, None]
    return jnp.concatenate([mx, mean], axis=1)


def kernel(xd1, xd2, xc1, xc2, xc3, xtc, W_gat1, a_src1, a_dst1, b_gat1, W_gcn1, b_gcn1, W_fcg1a, b_fcg1a, W_gat2, a_src2, a_dst2, b_gat2, W_gcn2, b_gcn2, W_fcg1b, b_fcg1b, W_fcg2, b_fcg2, W_cl1, b_cl1, W_cl2, b_cl2, W_fc1, b_fc1, W_fc2, b_fc2, W_out, b_out, edge_index1, batch_d1, edge_index2, batch_d2):
    d1 = _branch(xd1, edge_index1, batch_d1, W_gat1, a_src1, a_dst1, b_gat1,
                 W_gcn1, b_gcn1, W_fcg1a, b_fcg1a, W_fcg2, b_fcg2)
    d2 = _branch(xd2, edge_index2, batch_d2, W_gat2, a_src2, a_dst2, b_gat2,
                 W_gcn2, b_gcn2, W_fcg1b, b_fcg1b, W_fcg2, b_fcg2)
    xcl = _mm(jnp.concatenate([xc1, xc2, xc3, xtc], axis=1), W_cl1, b_cl1, act="relu")
    xcl = _mm(xcl, W_cl2, b_cl2, act="relu")
    xc = jnp.concatenate([d1, d2, xcl, xtc], axis=1)
    xc = _mm(xc, W_fc1, b_fc1, act="relu")
    xc = _mm(xc, W_fc2, b_fc2, act="relu")
    xc = jnp.concatenate([xc, xtc], axis=1)
    out = _mm(xc, W_out, b_out)
    return jnp.clip(out, -100.0, 100.0)


# speedup vs baseline: 1.0009x; 1.0009x over previous
import functools

import jax
import jax.numpy as jnp
from jax import lax
from jax.experimental import pallas as pl
from jax.experimental.pallas import tpu as pltpu
from jax.experimental.pallas import tpu_sc as plsc

N = 10000
B = 256
H = 10
C = 77
E_RAW = 160000
E_TOT = E_RAW + N
NP = 10240
EP = 172032
CHUNK = 128
TPA = EP // 32
TPBC = EP // 16
NCH_A = TPA // CHUNK
NCH_BC = TPBC // CHUNK
RPT = NP // 16

_MESH = plsc.VectorSubcoreMesh(core_axis_name="c", subcore_axis_name="s")


def _rup(x, m):
    return (x + m - 1) // m * m


def _mm_body(x_ref, w_ref, b_ref, o_ref, *, act):
    acc = jnp.dot(x_ref[...], w_ref[...], preferred_element_type=jnp.float32)
    acc = acc + b_ref[...]
    if act == "relu":
        acc = jnp.maximum(acc, 0.0)
    elif act == "elu":
        acc = jnp.where(acc > 0, acc, jnp.exp(acc) - 1.0)
    o_ref[...] = acc


def _mm(x, w, b, act="none", bm=1024, bn=1024):
    M, K = x.shape
    _, Nn = w.shape
    bm = min(bm, _rup(M, 8))
    bn = min(bn, _rup(Nn, 128))
    Mp, Kp, Np2 = _rup(M, bm), _rup(K, 128), _rup(Nn, bn)
    xp = jnp.pad(x, ((0, Mp - M), (0, Kp - K)))
    wp = jnp.pad(w, ((0, Kp - K), (0, Np2 - Nn)))
    bp = jnp.pad(b.reshape(1, -1), ((0, 0), (0, Np2 - Nn)))
    out = pl.pallas_call(
        functools.partial(_mm_body, act=act),
        grid=(Mp // bm, Np2 // bn),
        in_specs=[
            pl.BlockSpec((bm, Kp), lambda i, j: (i, 0)),
            pl.BlockSpec((Kp, bn), lambda i, j: (0, j)),
            pl.BlockSpec((1, bn), lambda i, j: (0, j)),
        ],
        out_specs=pl.BlockSpec((bm, bn), lambda i, j: (i, j)),
        out_shape=jax.ShapeDtypeStruct((Mp, Np2), jnp.float32),
    )(xp, wp, bp)
    return out[:M, :Nn]


def _edge_softmax_body(src_hbm, dst_hbm, es_hbm, ed_hbm,
                       ex_hbm, den0_hbm, den1_hbm,
                       sidx0, sidx1, didx0, didx1, esr0, esr1, edr0, edr1,
                       exb, zb, acc, si, sg):
    c = lax.axis_index("c")
    s = lax.axis_index("s")
    w = s * 2 + c
    lane = lax.iota(jnp.int32, 16)
    SIDX = (sidx0, sidx1)
    DIDX = (didx0, didx1)
    ESR = (esr0, esr1)
    EDR = (edr0, edr1)

    def idx_start(b, chv):
        base = w * TPA + chv * CHUNK
        pltpu.async_copy(src_hbm.at[pl.ds(base, CHUNK)], SIDX[b], si)
        pltpu.async_copy(dst_hbm.at[pl.ds(base, CHUNK)], DIDX[b], si)

    def idx_drain(b):
        pltpu.make_async_copy(src_hbm.at[pl.ds(0, CHUNK)], SIDX[b], si).wait()
        pltpu.make_async_copy(dst_hbm.at[pl.ds(0, CHUNK)], DIDX[b], si).wait()

    def gather_start(b):
        pltpu.async_copy(es_hbm.at[SIDX[b]], ESR[b], sg)
        pltpu.async_copy(ed_hbm.at[DIDX[b]], EDR[b], sg)

    def gather_drain(b):
        pltpu.make_async_copy(es_hbm.at[pl.ds(0, CHUNK)], ESR[b], sg).wait()
        pltpu.make_async_copy(ed_hbm.at[pl.ds(0, CHUNK)], EDR[b], sg).wait()

    def zrow(i, _):
        zb[i, :] = jnp.zeros((16,), jnp.float32)
        return 0
    lax.fori_loop(0, 16, zrow, 0)

    def zcp(j, _):
        pltpu.sync_copy(zb, acc.at[pl.ds(s * RPT + j * 16, 16)])
        return 0
    lax.fori_loop(0, RPT // 16, zcp, 0)
    plsc.subcore_barrier()

    def compute_scatter(b, chv):
        base = w * TPA + chv * CHUNK
        esr_b = ESR[b]
        edr_b = EDR[b]

        def edge(k, _):
            v = esr_b[k, :] + edr_b[k, :]
            v = jnp.maximum(v, 0.2 * v)
            v = jnp.exp(v)
            v = jnp.where(lane == 10, 1.0, v)
            exb[k, :] = v
            return 0
        lax.fori_loop(0, CHUNK, edge, 0, unroll=8)
        pltpu.sync_copy(exb, ex_hbm.at[pl.ds(base, CHUNK)])
        pltpu.sync_copy(exb, acc.at[DIDX[b]], add=True)

    idx_start(0, 0)
    idx_drain(0)
    gather_start(0)
    idx_start(1, 1)

    def grp(i, _):
        nxt = jnp.where(2 * i + 2 < NCH_A, 2 * i + 2, 0)
        nxt2 = jnp.where(2 * i + 3 < NCH_A, 2 * i + 3, 0)
        idx_drain(1)
        gather_drain(0)
        gather_start(1)
        compute_scatter(0, 2 * i)
        idx_start(0, nxt)
        idx_drain(0)
        gather_drain(1)
        gather_start(0)
        compute_scatter(1, 2 * i + 1)
        idx_start(1, nxt2)
        return 0
    lax.fori_loop(0, NCH_A // 2, grp, 0)
    idx_drain(1)
    gather_drain(0)
    plsc.subcore_barrier()

    @pl.when(c == 0)
    def _():
        pltpu.sync_copy(acc.at[pl.ds(s * RPT, RPT)],
                        den0_hbm.at[pl.ds(s * RPT, RPT)])

    @pl.when(c == 1)
    def _():
        pltpu.sync_copy(acc.at[pl.ds(s * RPT, RPT)],
                        den1_hbm.at[pl.ds(s * RPT, RPT)])


_edge_softmax = pl.kernel(
    _edge_softmax_body,
    out_type=(
        jax.ShapeDtypeStruct((EP, 16), jnp.float32),
        jax.ShapeDtypeStruct((NP, 16), jnp.float32),
        jax.ShapeDtypeStruct((NP, 16), jnp.float32),
    ),
    mesh=_MESH,
    compiler_params=pltpu.CompilerParams(use_tc_tiling_on_sc=False, needs_layout_passes=False),
    scratch_types=(
        pltpu.VMEM((CHUNK,), jnp.int32),
        pltpu.VMEM((CHUNK,), jnp.int32),
        pltpu.VMEM((CHUNK,), jnp.int32),
        pltpu.VMEM((CHUNK,), jnp.int32),
        pltpu.VMEM((CHUNK, 16), jnp.float32),
        pltpu.VMEM((CHUNK, 16), jnp.float32),
        pltpu.VMEM((CHUNK, 16), jnp.float32),
        pltpu.VMEM((CHUNK, 16), jnp.float32),
        pltpu.VMEM((CHUNK, 16), jnp.float32),
        pltpu.VMEM((16, 16), jnp.float32),
        pltpu.MemorySpace.VMEM_SHARED(shape=(NP, 16), dtype=jnp.float32),
        pltpu.SemaphoreType.DMA,
        pltpu.SemaphoreType.DMA,
    ),
)


def _gat_agg_body(src_hbm, dst_hbm, x_hbm, ex_hbm, agg_hbm,
                  sidx0, sidx1, didx0, didx1, xr0, xr1, exb0, exb1,
                  zb, acc, si, sg):
    c = lax.axis_index("c")
    s = lax.axis_index("s")
    SIDX = (sidx0, sidx1)
    DIDX = (didx0, didx1)
    XR = (xr0, xr1)
    EXB = (exb0, exb1)
    SI = (si, si)
    SG = (sg, sg)

    def idx_start(b, chv):
        base = s * TPBC + chv * CHUNK
        pltpu.async_copy(src_hbm.at[pl.ds(base, CHUNK)], SIDX[b], SI[b])
        pltpu.async_copy(dst_hbm.at[pl.ds(base, CHUNK)], DIDX[b], SI[b])
        pltpu.async_copy(ex_hbm.at[pl.ds(base, CHUNK)], EXB[b], SI[b])

    def idx_drain(b):
        pltpu.make_async_copy(src_hbm.at[pl.ds(0, CHUNK)], SIDX[b], SI[b]).wait()
        pltpu.make_async_copy(dst_hbm.at[pl.ds(0, CHUNK)], DIDX[b], SI[b]).wait()
        pltpu.make_async_copy(ex_hbm.at[pl.ds(0, CHUNK)], EXB[b], SI[b]).wait()

    def gather_start(b):
        pltpu.async_copy(x_hbm.at[SIDX[b]], XR[b], SG[b])

    def gather_drain(b):
        pltpu.make_async_copy(x_hbm.at[pl.ds(0, CHUNK)], XR[b], SG[b]).wait()

    def zrow(i, _):
        for g in range(5):
            zb[i, pl.ds(16 * g, 16)] = jnp.zeros((16,), jnp.float32)
        return 0
    lax.fori_loop(0, 16, zrow, 0)

    for p in range(5):
        def zcp(j, _):
            pltpu.sync_copy(zb, acc.at[pl.ds(s * RPT + j * 16, 16)])
            return 0
        lax.fori_loop(0, RPT // 16, zcp, 0)
        plsc.subcore_barrier()
        hl = c * 5 + p
        hlv = jnp.full((16,), hl, jnp.int32)

        def compute_scatter(b, hlv=hlv):
            xr_b = XR[b]
            exb_b = EXB[b]

            def edge(k, _):
                kv = jnp.full((16,), k, jnp.int32)
                wv = plsc.load_gather(exb_b, [kv, hlv])
                for g in range(5):
                    xr_b[k, pl.ds(16 * g, 16)] = xr_b[k, pl.ds(16 * g, 16)] * wv
                return 0
            lax.fori_loop(0, CHUNK, edge, 0, unroll=8)
            pltpu.sync_copy(xr_b, acc.at[DIDX[b]], add=True)

        idx_start(0, 0)
        idx_drain(0)
        gather_start(0)
        idx_start(1, 1)

        def grp(i, _):
            nxt = jnp.where(2 * i + 2 < NCH_BC, 2 * i + 2, 0)
            nxt2 = jnp.where(2 * i + 3 < NCH_BC, 2 * i + 3, 0)
            idx_drain(1)
            gather_drain(0)
            gather_start(1)
            compute_scatter(0)
            idx_start(0, nxt)
            idx_drain(0)
            gather_drain(1)
            gather_start(0)
            compute_scatter(1)
            idx_start(1, nxt2)
            return 0
        lax.fori_loop(0, NCH_BC // 2, grp, 0)
        idx_drain(1)
        gather_drain(0)
        plsc.subcore_barrier()

        @pl.when(c == 0)
        def _(p=p):
            pltpu.sync_copy(acc.at[pl.ds(s * RPT, RPT)],
                            agg_hbm.at[p, pl.ds(s * RPT, RPT)])

        @pl.when(c == 1)
        def _(p=p):
            pltpu.sync_copy(acc.at[pl.ds(s * RPT, RPT)],
                            agg_hbm.at[5 + p, pl.ds(s * RPT, RPT)])
        plsc.subcore_barrier()


_gat_agg = pl.kernel(
    _gat_agg_body,
    out_type=jax.ShapeDtypeStruct((H, NP, 80), jnp.float32),
    mesh=_MESH,
    compiler_params=pltpu.CompilerParams(use_tc_tiling_on_sc=False, needs_layout_passes=False),
    scratch_types=(
        pltpu.VMEM((CHUNK,), jnp.int32),
        pltpu.VMEM((CHUNK,), jnp.int32),
        pltpu.VMEM((CHUNK,), jnp.int32),
        pltpu.VMEM((CHUNK,), jnp.int32),
        pltpu.VMEM((CHUNK, 80), jnp.float32),
        pltpu.VMEM((CHUNK, 80), jnp.float32),
        pltpu.VMEM((CHUNK, 16), jnp.float32),
        pltpu.VMEM((CHUNK, 16), jnp.float32),
        pltpu.VMEM((16, 80), jnp.float32),
        pltpu.MemorySpace.VMEM_SHARED(shape=(NP, 80), dtype=jnp.float32),
        pltpu.SemaphoreType.DMA,
        pltpu.SemaphoreType.DMA,
    ),
)


def _gcn_agg_body(src_hbm, dst_hbm, h2_hbm, agg_hbm,
                  sidx0, sidx1, didx0, didx1, r0, r1, zb, acc,
                  si, sg):
    c = lax.axis_index("c")
    s = lax.axis_index("s")
    SIDX = (sidx0, sidx1)
    DIDX = (didx0, didx1)
    RR = (r0, r1)
    SI = (si, si)
    SG = (sg, sg)

    def idx_start(b, chv):
        base = s * TPBC + chv * CHUNK
        pltpu.async_copy(src_hbm.at[pl.ds(base, CHUNK)], SIDX[b], SI[b])
        pltpu.async_copy(dst_hbm.at[pl.ds(base, CHUNK)], DIDX[b], SI[b])

    def idx_drain_fix(b, blk):
        pltpu.make_async_copy(src_hbm.at[pl.ds(0, CHUNK)], SIDX[b], SI[b]).wait()
        pltpu.make_async_copy(dst_hbm.at[pl.ds(0, CHUNK)], DIDX[b], SI[b]).wait()
        if blk:
            for g in range(CHUNK // 16):
                SIDX[b][pl.ds(16 * g, 16)] = (
                    SIDX[b][pl.ds(16 * g, 16)] + blk * NP)

    def gather_start(b):
        pltpu.async_copy(h2_hbm.at[SIDX[b]], RR[b], SG[b])

    def gather_drain(b):
        pltpu.make_async_copy(h2_hbm.at[pl.ds(0, CHUNK)], RR[b], SG[b]).wait()

    def scat(b):
        pltpu.sync_copy(RR[b], acc.at[DIDX[b]], add=True)

    def zrow(i, _):
        for g in range(5):
            zb[i, pl.ds(16 * g, 16)] = jnp.zeros((16,), jnp.float32)
        return 0
    lax.fori_loop(0, 16, zrow, 0)

    for blk in range(H):
        owner = blk % 2

        @pl.when(c == owner)
        def _(blk=blk):
            def zcp(j, _):
                pltpu.sync_copy(zb, acc.at[pl.ds(s * RPT + j * 16, 16)])
                return 0
            lax.fori_loop(0, RPT // 16, zcp, 0)
            plsc.subcore_barrier()
            idx_start(0, 0)
            idx_drain_fix(0, blk)
            gather_start(0)
            idx_start(1, 1)

            def grp(i, _, blk=blk):
                nxt = jnp.where(2 * i + 2 < NCH_BC, 2 * i + 2, 0)
                nxt2 = jnp.where(2 * i + 3 < NCH_BC, 2 * i + 3, 0)
                idx_drain_fix(1, blk)
                gather_drain(0)
                gather_start(1)
                scat(0)
                idx_start(0, nxt)
                idx_drain_fix(0, blk)
                gather_drain(1)
                gather_start(0)
                scat(1)
                idx_start(1, nxt2)
                return 0
            lax.fori_loop(0, NCH_BC // 2, grp, 0)
            pltpu.make_async_copy(src_hbm.at[pl.ds(0, CHUNK)], sidx1, si).wait()
            pltpu.make_async_copy(dst_hbm.at[pl.ds(0, CHUNK)], didx1, si).wait()
            gather_drain(0)
            plsc.subcore_barrier()
            pltpu.sync_copy(acc.at[pl.ds(s * RPT, RPT)],
                            agg_hbm.at[blk, pl.ds(s * RPT, RPT)])
            plsc.subcore_barrier()


_gcn_agg = pl.kernel(
    _gcn_agg_body,
    out_type=jax.ShapeDtypeStruct((H, NP, 80), jnp.float32),
    mesh=_MESH,
    compiler_params=pltpu.CompilerParams(use_tc_tiling_on_sc=False, needs_layout_passes=False),
    scratch_types=(
        pltpu.VMEM((CHUNK,), jnp.int32),
        pltpu.VMEM((CHUNK,), jnp.int32),
        pltpu.VMEM((CHUNK,), jnp.int32),
        pltpu.VMEM((CHUNK,), jnp.int32),
        pltpu.VMEM((CHUNK, 80), jnp.float32),
        pltpu.VMEM((CHUNK, 80), jnp.float32),
        pltpu.VMEM((16, 80), jnp.float32),
        pltpu.MemorySpace.VMEM_SHARED(shape=(NP, 80), dtype=jnp.float32),
        pltpu.SemaphoreType.DMA,
        pltpu.SemaphoreType.DMA,
    ),
)


def _branch(x, edge_index, batch, Wg, a_s, a_d, bg, Wc, bc, W1, b1, W2, b2):
    loop = jnp.arange(N, dtype=jnp.int32)
    padi = jnp.full((EP - E_TOT,), N, jnp.int32)
    src = jnp.concatenate([edge_index[0].astype(jnp.int32), loop, padi])
    dst = jnp.concatenate([edge_index[1].astype(jnp.int32), loop, padi])

    W3 = Wg.reshape(C, H, C)
    As = jnp.einsum('chd,hd->ch', W3, a_s)
    Ad = jnp.einsum('chd,hd->ch', W3, a_d)
    AsAd = jnp.concatenate([
        jnp.pad(As, ((0, 0), (0, 6))), jnp.pad(Ad, ((0, 0), (0, 6)))], axis=1)
    esed = _mm(x, AsAd, jnp.zeros((32,), jnp.float32))
    es = jnp.pad(esed[:, :16], ((0, NP - N), (0, 0)))
    ed = jnp.pad(esed[:, 16:], ((0, NP - N), (0, 0)))

    ex, den0, den1 = _edge_softmax(src, dst, es, ed)
    den = den0[:N] + den1[:N]
    deg = den[:, 10]
    dinv = deg ** -0.5

    xpad = jnp.pad(x, ((0, NP - N), (0, 80 - C)))
    agg = _gat_agg(src, dst, xpad, ex)
    aggt = agg[:, :N, :].transpose(1, 0, 2) / (den[:, :10, None] + 1e-16)
    W_bd = jax.scipy.linalg.block_diag(
        *[jnp.pad(W3[:, h, :], ((0, 3), (0, 0))) for h in range(H)])
    gat = _mm(aggt.reshape(N, H * 80), W_bd, bg, act="elu")

    y = gat * dinv[:, None]
    h2 = _mm(y, Wc, jnp.zeros((H * C,), jnp.float32))
    h2r = jnp.pad(h2, ((0, NP - N), (0, 30))).reshape(NP, H, 80).transpose(1, 0, 2)
    aggc = _gcn_agg(src, dst, h2r.reshape(H * NP, 80))
    back = aggc.transpose(1, 0, 2).reshape(NP, H * 80)[:N, :H * C]
    z = jnp.maximum(back * dinv[:, None] + bc, 0.0)

    pooled = _pool(z, batch)
    z = _mm(pooled, W1, b1, act="relu")
    z = _mm(z, W2, b2, act="relu")
    return z


def _pool(x, batch):
    batch = batch.astype(jnp.int32)
    mx = jax.ops.segment_max(x, batch, num_segments=B)
    mx = jnp.where(jnp.isfinite(mx), mx, 0.0)
    x1 = jnp.concatenate([x, jnp.ones((x.shape[0], 1), x.dtype)], axis=1)
    pt = (jnp.arange(B, dtype=jnp.int32)[:, None] == batch[None, :]).astype(jnp.float32)
    kc = 2500
    se = sum(_mm(pt[:, k:k + kc], x1[k:k + kc],
                 jnp.zeros((x1.shape[1],), jnp.float32))
             for k in range(0, N, kc))
    cnt = se[:, -1]
    mean = se[:, :-1] / jnp.maximum(cnt, 1.0)[:, None]
    return jnp.concatenate([mx, mean], axis=1)


def kernel(xd1, xd2, xc1, xc2, xc3, xtc, W_gat1, a_src1, a_dst1, b_gat1, W_gcn1, b_gcn1, W_fcg1a, b_fcg1a, W_gat2, a_src2, a_dst2, b_gat2, W_gcn2, b_gcn2, W_fcg1b, b_fcg1b, W_fcg2, b_fcg2, W_cl1, b_cl1, W_cl2, b_cl2, W_fc1, b_fc1, W_fc2, b_fc2, W_out, b_out, edge_index1, batch_d1, edge_index2, batch_d2):
    d1 = _branch(xd1, edge_index1, batch_d1, W_gat1, a_src1, a_dst1, b_gat1,
                 W_gcn1, b_gcn1, W_fcg1a, b_fcg1a, W_fcg2, b_fcg2)
    d2 = _branch(xd2, edge_index2, batch_d2, W_gat2, a_src2, a_dst2, b_gat2,
                 W_gcn2, b_gcn2, W_fcg1b, b_fcg1b, W_fcg2, b_fcg2)
    xcl = _mm(jnp.concatenate([xc1, xc2, xc3, xtc], axis=1), W_cl1, b_cl1, act="relu")
    xcl = _mm(xcl, W_cl2, b_cl2, act="relu")
    xc = jnp.concatenate([d1, d2, xcl, xtc], axis=1)
    xc = _mm(xc, W_fc1, b_fc1, act="relu")
    xc = _mm(xc, W_fc2, b_fc2, act="relu")
    xc = jnp.concatenate([xc, xtc], axis=1)
    out = _mm(xc, W_out, b_out)
    return jnp.clip(out, -100.0, 100.0)
